# baseline (speedup 1.0000x reference)
import jax
import jax.numpy as jnp
from jax.experimental import pallas as pl
from jax.experimental.pallas import tpu as pltpu

R_MAX = 32
S_MAX = 2
N_RES_BINS = 2 * R_MAX + 2
N_CHAIN_BINS = 2 * S_MAX + 2
NO_BINS = N_RES_BINS + N_RES_BINS + 1 + N_CHAIN_BINS
C_Z = 128

TI = 32
TJ = 1024
GRP = 4
M2 = TI // GRP
L = GRP * TJ

SEG_R = 72
SEG_T = 72
SEG_C = 8
NB_PAD = SEG_R + SEG_T + SEG_C


def _body(asym_i, res_i, ent_i, tok_i, sym_i,
          asym_j, res_j, ent_j, tok_j, sym_j,
          wt_ref, o_ref):
    ai = asym_i[...].astype(jnp.int32)
    ri = res_i[...].astype(jnp.int32)
    ei = ent_i[...].astype(jnp.int32)
    ki = tok_i[...].astype(jnp.int32)
    si = sym_i[...].astype(jnp.int32)
    aj = asym_j[0]
    rj = res_j[0]
    ej = ent_j[0]
    kj = tok_j[0]
    sj = sym_j[0]

    same_chain = ai == aj
    same_res = ri == rj

    r = jnp.where(same_chain,
                  jnp.clip(ri - rj + R_MAX, 0, 2 * R_MAX),
                  2 * R_MAX + 1)
    t = jnp.where(same_chain & same_res,
                  jnp.clip(ki - kj + R_MAX, 0, 2 * R_MAX),
                  2 * R_MAX + 1)
    e = (ei == ej)
    c = jnp.where(e,
                  jnp.clip(si - sj + S_MAX, 0, 2 * S_MAX),
                  2 * S_MAX + 1)

    r3 = r.reshape(M2, 1, L)
    t3 = t.reshape(M2, 1, L)
    c3 = c.reshape(M2, 1, L)
    e3 = e.reshape(M2, 1, L)

    kr = jax.lax.broadcasted_iota(jnp.int32, (1, SEG_R, 1), 1)
    kt = jax.lax.broadcasted_iota(jnp.int32, (1, SEG_T, 1), 1)
    kc = jax.lax.broadcasted_iota(jnp.int32, (1, SEG_C, 1), 1)

    fr = (kr == r3).astype(jnp.bfloat16)
    ft = (kt == t3).astype(jnp.bfloat16)
    fc = ((kc == c3) | ((kc == N_CHAIN_BINS) & e3)).astype(jnp.bfloat16)

    feat = jnp.concatenate([fr, ft, fc], axis=1)

    wt_b = jnp.broadcast_to(wt_ref[...], (M2, NB_PAD, C_Z))
    acc = jax.lax.dot_general(
        feat, wt_b,
        dimension_numbers=(((1,), (1,)), ((0,), (0,))),
        preferred_element_type=jnp.float32)
    o_ref[...] = acc.reshape(1, TI, TJ, C_Z)


@jax.jit
def kernel(asym_id, residue_index, entity_id, token_index, sym_id, W):
    B, N = asym_id.shape
    ni, nj = N // TI, N // TJ

    wt_full = W.T.astype(jnp.bfloat16)
    wt = jnp.zeros((NB_PAD, C_Z), jnp.bfloat16)
    wt = wt.at[0:N_RES_BINS].set(wt_full[0:N_RES_BINS])
    wt = wt.at[SEG_R:SEG_R + N_RES_BINS].set(
        wt_full[N_RES_BINS:2 * N_RES_BINS])
    wt = wt.at[SEG_R + SEG_T:SEG_R + SEG_T + N_CHAIN_BINS].set(
        wt_full[2 * N_RES_BINS + 1:NO_BINS])
    wt = wt.at[SEG_R + SEG_T + N_CHAIN_BINS].set(
        wt_full[2 * N_RES_BINS])

    def expand_i(a):
        return jnp.broadcast_to(
            a.astype(jnp.int16).reshape(ni * M2, GRP, 1),
            (ni * M2, GRP, TJ)).reshape(ni * M2, L)

    def expand_j(a):
        return jnp.broadcast_to(
            a.reshape(nj, 1, 1, TJ), (nj, 1, GRP, TJ)).reshape(nj, 1, L)

    arrays = (asym_id, residue_index, entity_id, token_index, sym_id)
    i_in = [expand_i(a) for a in arrays]
    j_in = [expand_j(a) for a in arrays]

    i_spec = pl.BlockSpec((M2, L), lambda i, j: (i, 0))
    j_spec = pl.BlockSpec((1, 1, L), lambda i, j: (j, 0, 0))
    w_spec = pl.BlockSpec((NB_PAD, C_Z), lambda i, j: (0, 0))

    out = pl.pallas_call(
        _body,
        grid=(ni, nj),
        in_specs=[i_spec] * 5 + [j_spec] * 5 + [w_spec],
        out_specs=pl.BlockSpec((1, TI, TJ, C_Z), lambda i, j: (0, i, j, 0)),
        out_shape=jax.ShapeDtypeStruct((B, N, N, C_Z), jnp.float32),
        compiler_params=pltpu.CompilerParams(
            dimension_semantics=("parallel", "parallel"),
        ),
    )(*i_in, *j_in, wt)
    return out


# speedup vs baseline: 1.0259x; 1.0259x over previous
import jax
import jax.numpy as jnp
from jax.experimental import pallas as pl
from jax.experimental.pallas import tpu as pltpu

R_MAX = 32
S_MAX = 2
N_RES_BINS = 2 * R_MAX + 2
N_CHAIN_BINS = 2 * S_MAX + 2
NO_BINS = N_RES_BINS + N_RES_BINS + 1 + N_CHAIN_BINS
C_Z = 128

TI = 128
TJ = 256
GRP = 16
M2 = TI // GRP
L = GRP * TJ

SEG_R = 72
SEG_T = 72
SEG_C = 8
NB_PAD = SEG_R + SEG_T + SEG_C


def _body(asym_i, res_i, ent_i, tok_i, sym_i,
          asym_j, res_j, ent_j, tok_j, sym_j,
          wt_ref, o_ref):
    ai = asym_i[...].astype(jnp.int32)
    ri = res_i[...].astype(jnp.int32)
    ei = ent_i[...].astype(jnp.int32)
    ki = tok_i[...].astype(jnp.int32)
    si = sym_i[...].astype(jnp.int32)
    aj = asym_j[0]
    rj = res_j[0]
    ej = ent_j[0]
    kj = tok_j[0]
    sj = sym_j[0]

    same_chain = ai == aj
    same_res = ri == rj

    r = jnp.where(same_chain,
                  jnp.clip(ri - rj + R_MAX, 0, 2 * R_MAX),
                  2 * R_MAX + 1)
    t = jnp.where(same_chain & same_res,
                  jnp.clip(ki - kj + R_MAX, 0, 2 * R_MAX),
                  2 * R_MAX + 1)
    e = (ei == ej)
    c = jnp.where(e,
                  jnp.clip(si - sj + S_MAX, 0, 2 * S_MAX),
                  2 * S_MAX + 1)

    r3 = r.reshape(M2, 1, L)
    t3 = t.reshape(M2, 1, L)
    c3 = c.reshape(M2, 1, L)
    e3 = e.reshape(M2, 1, L)

    kr = jax.lax.broadcasted_iota(jnp.int32, (1, SEG_R, 1), 1)
    kt = jax.lax.broadcasted_iota(jnp.int32, (1, SEG_T, 1), 1)
    kc = jax.lax.broadcasted_iota(jnp.int32, (1, SEG_C, 1), 1)

    fr = (kr == r3).astype(jnp.bfloat16)
    ft = (kt == t3).astype(jnp.bfloat16)
    fc = ((kc == c3) | ((kc == N_CHAIN_BINS) & e3)).astype(jnp.bfloat16)

    feat = jnp.concatenate([fr, ft, fc], axis=1)

    wt_b = jnp.broadcast_to(wt_ref[...], (M2, NB_PAD, C_Z))
    acc = jax.lax.dot_general(
        feat, wt_b,
        dimension_numbers=(((1,), (1,)), ((0,), (0,))),
        preferred_element_type=jnp.float32)
    o_ref[...] = acc.reshape(1, TI, TJ, C_Z)


@jax.jit
def kernel(asym_id, residue_index, entity_id, token_index, sym_id, W):
    B, N = asym_id.shape
    ni, nj = N // TI, N // TJ

    wt_full = W.T.astype(jnp.bfloat16)
    wt = jnp.zeros((NB_PAD, C_Z), jnp.bfloat16)
    wt = wt.at[0:N_RES_BINS].set(wt_full[0:N_RES_BINS])
    wt = wt.at[SEG_R:SEG_R + N_RES_BINS].set(
        wt_full[N_RES_BINS:2 * N_RES_BINS])
    wt = wt.at[SEG_R + SEG_T:SEG_R + SEG_T + N_CHAIN_BINS].set(
        wt_full[2 * N_RES_BINS + 1:NO_BINS])
    wt = wt.at[SEG_R + SEG_T + N_CHAIN_BINS].set(
        wt_full[2 * N_RES_BINS])

    def expand_i(a):
        return jnp.broadcast_to(
            a.astype(jnp.int16).reshape(ni * M2, GRP, 1),
            (ni * M2, GRP, TJ)).reshape(ni * M2, L)

    def expand_j(a):
        return jnp.broadcast_to(
            a.reshape(nj, 1, 1, TJ), (nj, 1, GRP, TJ)).reshape(nj, 1, L)

    arrays = (asym_id, residue_index, entity_id, token_index, sym_id)
    i_in = [expand_i(a) for a in arrays]
    j_in = [expand_j(a) for a in arrays]

    i_spec = pl.BlockSpec((M2, L), lambda i, j: (i, 0))
    j_spec = pl.BlockSpec((1, 1, L), lambda i, j: (j, 0, 0))
    w_spec = pl.BlockSpec((NB_PAD, C_Z), lambda i, j: (0, 0))

    out = pl.pallas_call(
        _body,
        grid=(ni, nj),
        in_specs=[i_spec] * 5 + [j_spec] * 5 + [w_spec],
        out_specs=pl.BlockSpec((1, TI, TJ, C_Z), lambda i, j: (0, i, j, 0)),
        out_shape=jax.ShapeDtypeStruct((B, N, N, C_Z), jnp.float32),
        compiler_params=pltpu.CompilerParams(
            dimension_semantics=("parallel", "parallel"),
        ),
    )(*i_in, *j_in, wt)
    return out
